# baseline (speedup 1.0000x reference)
import functools

import jax
import jax.numpy as jnp
import numpy as np
from jax.experimental import pallas as pl
from jax.experimental.pallas import tpu as pltpu

_BINS = 10
_EDGES64 = (np.arange(11, dtype=np.float64) / 10.0).astype(np.float32).astype(np.float64)
_THRESH = [float(np.log(e / (1.0 - e))) for e in _EDGES64[1:10]]


def _ghm_body(pred_ref, tgt_ref, lw_ref, out_ref, acc_ref, *, nsteps, n_total):
    i = pl.program_id(0)

    @pl.when(i == 0)
    def _init():
        acc_ref[...] = jnp.zeros_like(acc_ref)

    p = pred_ref[...]
    t = tgt_ref[...].astype(jnp.float32)
    x = p * (1.0 - 2.0 * t)
    bce = jnp.maximum(p, 0.0) - p * t + jnp.log1p(jnp.exp(-jnp.abs(p)))

    acc_ref[0:1, :] += jnp.sum(bce, axis=0, keepdims=True)
    for k, c in enumerate(_THRESH):
        m = x >= c
        acc_ref[1 + k:2 + k, :] += jnp.sum(
            jnp.where(m, 1.0, 0.0), axis=0, keepdims=True)
        acc_ref[10 + k:11 + k, :] += jnp.sum(
            jnp.where(m, bce, 0.0), axis=0, keepdims=True)

    @pl.when(i == nsteps - 1)
    def _fin():
        lw = lw_ref[0, 0]
        validf = jnp.where(lw > 0.0, jnp.float32(1.0), jnp.float32(0.0))
        zero = jnp.float32(0.0)
        cum_cnt = [jnp.float32(n_total)]
        cum_bce = [jnp.sum(acc_ref[0:1, :])]
        for k in range(9):
            cum_cnt.append(jnp.sum(acc_ref[1 + k:2 + k, :]))
            cum_bce.append(jnp.sum(acc_ref[10 + k:11 + k, :]))
        cum_cnt.append(zero)
        cum_bce.append(zero)
        loss_sum = zero
        n = zero
        for j in range(_BINS):
            cnt = cum_cnt[j] - cum_cnt[j + 1]
            s = cum_bce[j] - cum_bce[j + 1]
            nz = cnt > 0.0
            n += jnp.where(nz, 1.0, 0.0)
            loss_sum += jnp.where(nz, s / jnp.maximum(cnt, 1.0), 0.0)
        loss = jnp.where(n > 0.0, loss_sum / jnp.maximum(n, 1.0), 0.0) * validf
        out_ref[0, 0] = loss


def kernel(pred, target, label_weight):
    rows, cols = pred.shape
    block = 2048
    while rows % block:
        block //= 2
    nsteps = rows // block
    out = pl.pallas_call(
        functools.partial(_ghm_body, nsteps=nsteps, n_total=float(rows * cols)),
        grid=(nsteps,),
        in_specs=[
            pl.BlockSpec((block, cols), lambda i: (i, 0)),
            pl.BlockSpec((block, cols), lambda i: (i, 0)),
            pl.BlockSpec(memory_space=pltpu.SMEM),
        ],
        out_specs=pl.BlockSpec(memory_space=pltpu.SMEM),
        out_shape=jax.ShapeDtypeStruct((1, 1), jnp.float32),
        scratch_shapes=[pltpu.VMEM((19, cols), jnp.float32)],
        compiler_params=pltpu.CompilerParams(dimension_semantics=("arbitrary",)),
    )(pred, target.astype(jnp.int32), label_weight)
    return out[0, 0]
